# baseline (speedup 1.0000x reference)
import functools

import jax
import jax.numpy as jnp
from jax import lax
from jax.experimental import pallas as pl
from jax.experimental.pallas import tpu as pltpu
from jax.experimental.pallas import tpu_sc as plsc

N_NODES = 10000
N_PAD = 10112
D_FEAT = 128
D_EDGE = 16

NC = 2
NS = 16
NW = NC * NS

CHUNK = 128
GROUP = 8
ROWS_PER_TILE = N_PAD // NS


def _edge_weights(phys, e_pad):
    blk_e = 4096
    n_rows = e_pad // 128
    grid = n_rows // 32
    last = phys.shape[0] // blk_e - (phys.shape[0] % blk_e == 0)

    def wk(p_ref, o_ref):
        x = p_ref[...].reshape(32, 128, D_EDGE)
        o_ref[...] = jnp.sum(x, axis=-1)

    return pl.pallas_call(
        wk,
        grid=(grid,),
        in_specs=[pl.BlockSpec((blk_e, D_EDGE),
                               lambda i: (jnp.minimum(i, last), 0))],
        out_specs=pl.BlockSpec((32, 128), lambda i: (i, 0)),
        out_shape=jax.ShapeDtypeStruct((n_rows, 128), jnp.float32),
    )(phys)


def _sc_aggregate(features, src2d, dst2d, w2d, n_edges, edges_per_tile):
    n_chunks = edges_per_tile // CHUNK
    n_groups = n_chunks // GROUP

    mesh = plsc.VectorSubcoreMesh(core_axis_name="c", subcore_axis_name="s")

    @functools.partial(
        pl.kernel,
        mesh=mesh,
        out_type=jax.ShapeDtypeStruct((NC, N_PAD, D_FEAT), jnp.float32),
        scratch_types=[
            pltpu.VMEM((CHUNK, D_FEAT), jnp.float32),
            pltpu.VMEM((GROUP, 128), jnp.float32),
            pltpu.VMEM((GROUP, 128), jnp.int32),
            pltpu.VMEM((GROUP, 128), jnp.int32),
            pltpu.SemaphoreType.DMA,
            pltpu.VMEM_SHARED((N_PAD, D_FEAT), jnp.float32),
        ],
    )
    def k(feat_hbm, src_hbm, dst_hbm, w_hbm, out_hbm,
          gbuf, w_v, src_v, dst_v, gsem, acc_sh):
        c = lax.axis_index("c")
        s = lax.axis_index("s")
        wid = s * NC + c
        edge_base = wid * edges_per_tile
        idx_base = pl.multiple_of(edge_base // 128, 8)
        nv = jnp.clip((n_edges - edge_base) // CHUNK, 0, n_chunks)

        zeros16 = jnp.zeros((16,), jnp.float32)

        def zero_body(r, _):
            for g in range(D_FEAT // 16):
                gbuf[r, pl.ds(g * 16, 16)] = zeros16
            return 0
        lax.fori_loop(0, CHUNK, zero_body, 0)

        base_row = pl.multiple_of(s * ROWS_PER_TILE, 8)
        done = 0
        while done < ROWS_PER_TILE:
            n = min(CHUNK, ROWS_PER_TILE - done)
            pltpu.sync_copy(gbuf.at[pl.ds(0, n)],
                            acc_sh.at[pl.ds(base_row + done, n)])
            done += n
        plsc.subcore_barrier()

        def load_idx(g):
            row0 = pl.multiple_of(idx_base + g * GROUP, 8)
            pltpu.sync_copy(src_hbm.at[pl.ds(row0, GROUP)], src_v)
            pltpu.sync_copy(dst_hbm.at[pl.ds(row0, GROUP)], dst_v)
            pltpu.sync_copy(w_hbm.at[pl.ds(row0, GROUP)], w_v)

        def scale(r):
            def scale_body(gi, _):
                off = pl.multiple_of(gi * 16, 16)
                w16 = w_v[r, pl.ds(off, 16)]
                for e in range(16):
                    row = off + e
                    w = w16[e]
                    for g in range(D_FEAT // 16):
                        sl = pl.ds(g * 16, 16)
                        gbuf[row, sl] = gbuf[row, sl] * w
                return 0
            lax.fori_loop(0, CHUNK // 16, scale_body, 0)

        def chunk_body(j, _):
            @pl.when(j % GROUP == 0)
            def _():
                load_idx(j // GROUP)

            @pl.when(j < nv)
            def _():
                r = j % GROUP
                pltpu.async_copy(feat_hbm.at[src_v.at[r]], gbuf, gsem)
                pltpu.make_async_copy(feat_hbm.at[pl.ds(0, CHUNK)], gbuf,
                                      gsem).wait()
                scale(r)
                pltpu.sync_copy(gbuf, acc_sh.at[dst_v.at[r]], add=True)
            return 0

        lax.fori_loop(0, n_chunks, chunk_body, 0)
        plsc.subcore_barrier()

        pltpu.sync_copy(acc_sh.at[pl.ds(base_row, ROWS_PER_TILE)],
                        out_hbm.at[c, pl.ds(base_row, ROWS_PER_TILE)])

    return k(features, src2d, dst2d, w2d)


def _combine_partials(partials):
    blk = 1000

    def add_k(p_ref, o_ref):
        o_ref[...] = p_ref[0] + p_ref[1]

    return pl.pallas_call(
        add_k,
        grid=(N_NODES // blk,),
        in_specs=[pl.BlockSpec((NC, blk, D_FEAT), lambda i: (0, i, 0))],
        out_specs=pl.BlockSpec((blk, D_FEAT), lambda i: (i, 0)),
        out_shape=jax.ShapeDtypeStruct((N_NODES, D_FEAT), jnp.float32),
    )(partials)


@jax.jit
def kernel(features, adjacency_list, physics_features):
    n_edges = adjacency_list.shape[1]
    align = GROUP * 128
    edges_per_tile = -(-n_edges // (NW * align)) * align
    e_pad = edges_per_tile * NW
    pad = e_pad - n_edges

    src = adjacency_list[0].astype(jnp.int32)
    dst = adjacency_list[1].astype(jnp.int32)
    if pad:
        src = jnp.concatenate([src, jnp.zeros((pad,), jnp.int32)])
        dst = jnp.concatenate([dst, jnp.zeros((pad,), jnp.int32)])

    src2d = src.reshape(e_pad // 128, 128)
    dst2d = dst.reshape(e_pad // 128, 128)
    w2d = _edge_weights(physics_features.astype(jnp.float32), e_pad)

    partials = _sc_aggregate(features, src2d, dst2d, w2d,
                             n_edges, edges_per_tile)
    return _combine_partials(partials)
